# baseline (speedup 1.0000x reference)
import functools

import jax
import jax.numpy as jnp
from jax import lax
from jax.experimental import pallas as pl
from jax.experimental.pallas import tpu as pltpu

Q = 1024
D = 128
K = 100000
KB = 2048
KPAD = 100352
NRET = 8


def _dist_body(q_ref, k_ref, d2_ref):
    q = q_ref[...]
    kblk = k_ref[...]
    qsq = jnp.sum(q * q, axis=1, keepdims=True)
    ksq = jnp.sum(kblk * kblk, axis=1)[None, :]
    qk = lax.dot_general(q, kblk, (((1,), (1,)), ((), ())),
                         preferred_element_type=jnp.float32)
    d2 = jnp.maximum(qsq + ksq - 2.0 * qk, 0.0)
    d2_ref[...] = d2


def _distances(q, k):
    kp = jnp.pad(k, ((0, KPAD - K), (0, 0)), constant_values=1e4)
    grid = (KPAD // KB,)
    return pl.pallas_call(
        _dist_body,
        grid=grid,
        in_specs=[
            pl.BlockSpec((Q, D), lambda i: (0, 0)),
            pl.BlockSpec((KB, D), lambda i: (i, 0)),
        ],
        out_specs=pl.BlockSpec((Q, KB), lambda i: (0, i)),
        out_shape=jax.ShapeDtypeStruct((Q, KPAD), jnp.float32),
    )(q, kp)


def kernel(q, k, obs):
    d2 = _distances(q, k)
    _, idx = lax.top_k(-d2, NRET)
    sel = obs[idx]
    return jnp.transpose(sel, (1, 0, 2, 3, 4))


# speedup vs baseline: 2.0083x; 2.0083x over previous
import functools

import jax
import jax.numpy as jnp
from jax import lax
from jax.experimental import pallas as pl
from jax.experimental.pallas import tpu as pltpu

Q = 1024
D = 128
K = 100000
KB = 2048
KPAD = 100352
NRET = 8
SEGW = 16
NSEG = KPAD // SEGW
SEGB = KB // SEGW
NSEL = 16
QB = 256


def _dist_body(q_ref, k_ref, d2t_ref, smin_ref, qsq_ref):
    i = pl.program_id(0)

    @pl.when(i == 0)
    def _():
        qs = jnp.sum(q_ref[...] * q_ref[...], axis=1)
        qsq_ref[...] = qs[None, :]

    kblk = k_ref[...]
    ksq = jnp.sum(kblk * kblk, axis=1, keepdims=True)
    qkt = lax.dot_general(kblk, q_ref[...], (((1,), (1,)), ((), ())),
                          preferred_element_type=jnp.float32)
    d2t = jnp.maximum(qsq_ref[...] + ksq - 2.0 * qkt, 0.0)
    d2t_ref[...] = d2t
    smin_ref[...] = jnp.min(d2t.reshape(SEGB, SEGW, Q), axis=1)


def _distances(q, k):
    kp = jnp.pad(k, ((0, KPAD - K), (0, 0)), constant_values=1e4)
    grid = (KPAD // KB,)
    return pl.pallas_call(
        _dist_body,
        grid=grid,
        in_specs=[
            pl.BlockSpec((Q, D), lambda i: (0, 0)),
            pl.BlockSpec((KB, D), lambda i: (i, 0)),
        ],
        out_specs=[
            pl.BlockSpec((KB, Q), lambda i: (i, 0)),
            pl.BlockSpec((SEGB, Q), lambda i: (i, 0)),
        ],
        out_shape=[
            jax.ShapeDtypeStruct((KPAD, Q), jnp.float32),
            jax.ShapeDtypeStruct((NSEG, Q), jnp.float32),
        ],
        scratch_shapes=[pltpu.VMEM((1, Q), jnp.float32)],
    )(q, kp)


def _segsel_body(smin_ref, ids_ref):
    s = smin_ref[...]
    iota = lax.broadcasted_iota(jnp.int32, s.shape, 0)
    ids = []
    for _ in range(NSEL):
        m = jnp.min(s, axis=0, keepdims=True)
        am = jnp.min(jnp.where(s == m, iota, jnp.int32(2**30)),
                     axis=0, keepdims=True)
        ids.append(am[0, :])
        s = jnp.where(iota == am, jnp.float32(jnp.inf), s)
    ids = jnp.stack(ids, axis=0)
    rank = jnp.sum((ids[None, :, :] < ids[:, None, :]).astype(jnp.int32),
                   axis=1)
    slot = lax.broadcasted_iota(jnp.int32, (NSEL, NSEL, QB), 0)
    placed = jnp.where(rank[None, :, :] == slot, ids[None, :, :], 0)
    ids_ref[...] = jnp.sum(placed, axis=1)


def _select_segments(smin):
    return pl.pallas_call(
        _segsel_body,
        grid=(Q // QB,),
        in_specs=[pl.BlockSpec((NSEG, QB), lambda i: (0, i))],
        out_specs=pl.BlockSpec((NSEL, QB), lambda i: (0, i)),
        out_shape=jax.ShapeDtypeStruct((NSEL, Q), jnp.int32),
    )(smin)


def kernel(q, k, obs):
    d2t, smin = _distances(q, k)
    segids = _select_segments(smin)
    rows = (segids.T[:, :, None] * SEGW
            + jnp.arange(SEGW, dtype=jnp.int32)[None, None, :]
            ).reshape(Q, NSEL * SEGW)
    cand = d2t[rows, jnp.arange(Q, dtype=jnp.int32)[:, None]]
    _, cpos = lax.top_k(-cand, NRET)
    idx = jnp.take_along_axis(rows, cpos, axis=1)
    sel = obs[idx]
    return jnp.transpose(sel, (1, 0, 2, 3, 4))
